# baseline (speedup 1.0000x reference)
import functools

import jax
import jax.numpy as jnp
from jax import lax
from jax.experimental import pallas as pl
from jax.experimental.pallas import tpu as pltpu
from jax.experimental.pallas import tpu_sc as plsc

N_FEATS = 32
BATCH = 16384
NUM_CORES = 2
NUM_SUBCORES = 16
LANES = 16
NUM_WORKERS = NUM_CORES * NUM_SUBCORES
BPW = BATCH // NUM_WORKERS
IDX_CHUNK = 128
N_CHUNKS = BPW // IDX_CHUNK


def _mf_body(user_hbm, item_hbm, gb_hbm, ub_hbm, ib_hbm, ue_hbm, ie_hbm,
             out_hbm,
             uidx_v, iidx_v, ucols_v, icols_v, ub_v, ib_v, out_v, gb_v,
             sem):
    wid = lax.axis_index("s") * NUM_CORES + lax.axis_index("c")
    base = wid * BPW

    pltpu.sync_copy(gb_hbm, gb_v)
    for k in range(N_CHUNKS):
        off = base + k * IDX_CHUNK
        pltpu.sync_copy(user_hbm.at[pl.ds(off, IDX_CHUNK)], uidx_v.at[k])
        pltpu.sync_copy(item_hbm.at[pl.ds(off, IDX_CHUNK)], iidx_v.at[k])

    for k in range(N_CHUNKS):
        dst = pl.ds(k * IDX_CHUNK, IDX_CHUNK)
        pltpu.async_copy(ub_hbm.at[0].at[uidx_v.at[k]], ub_v.at[dst], sem)
        pltpu.async_copy(ib_hbm.at[0].at[iidx_v.at[k]], ib_v.at[dst], sem)
        for j in range(N_FEATS):
            pltpu.async_copy(ue_hbm.at[j].at[uidx_v.at[k]],
                             ucols_v.at[j].at[dst], sem)
            pltpu.async_copy(ie_hbm.at[j].at[iidx_v.at[k]],
                             icols_v.at[j].at[dst], sem)

    pltpu.make_async_copy(ub_hbm.at[0, pl.ds(0, BPW)], ub_v, sem).wait()
    pltpu.make_async_copy(ib_hbm.at[0, pl.ds(0, BPW)], ib_v, sem).wait()
    pltpu.make_async_copy(ue_hbm.at[:, pl.ds(0, BPW)], ucols_v, sem).wait()
    pltpu.make_async_copy(ie_hbm.at[:, pl.ds(0, BPW)], icols_v, sem).wait()

    g = gb_v[...]

    def chunk_body(c, _):
        sl = pl.ds(c * LANES, LANES)
        acc = ub_v[sl] + ib_v[sl] + g
        for j in range(N_FEATS):
            acc = acc + ucols_v[j, sl] * icols_v[j, sl]
        out_v[sl] = acc
        return _

    lax.fori_loop(0, BPW // LANES, chunk_body, None)

    pltpu.sync_copy(out_v, out_hbm.at[pl.ds(base, BPW)])


_mf_kernel = pl.kernel(
    _mf_body,
    out_type=jax.ShapeDtypeStruct((BATCH,), jnp.float32),
    mesh=plsc.VectorSubcoreMesh(core_axis_name="c", subcore_axis_name="s",
                                num_cores=NUM_CORES,
                                num_subcores=NUM_SUBCORES),
    scratch_types=[
        pltpu.VMEM((N_CHUNKS, IDX_CHUNK), jnp.int32),
        pltpu.VMEM((N_CHUNKS, IDX_CHUNK), jnp.int32),
        pltpu.VMEM((N_FEATS, BPW), jnp.float32),
        pltpu.VMEM((N_FEATS, BPW), jnp.float32),
        pltpu.VMEM((BPW,), jnp.float32),
        pltpu.VMEM((BPW,), jnp.float32),
        pltpu.VMEM((BPW,), jnp.float32),
        pltpu.VMEM((LANES,), jnp.float32),
        pltpu.SemaphoreType.DMA,
    ],
    compiler_params=pltpu.CompilerParams(needs_layout_passes=False,
                                         use_tc_tiling_on_sc=False),
)


@jax.jit
def kernel(user, item, g_bias, u_bias_w, i_bias_w, u_embed_w, i_embed_w):
    gb = jnp.full((LANES,), g_bias, jnp.float32)
    ub = u_bias_w.T
    ib = i_bias_w.T
    ue = u_embed_w.T
    ie = i_embed_w.T
    return _mf_kernel(user, item, gb, ub, ib, ue, ie)


# speedup vs baseline: 6.0170x; 6.0170x over previous
import functools

import jax
import jax.numpy as jnp
from jax import lax
from jax.experimental import pallas as pl
from jax.experimental.pallas import tpu as pltpu
from jax.experimental.pallas import tpu_sc as plsc

N_FEATS = 32
BATCH = 16384
NUM_CORES = 2
NUM_SUBCORES = 16
LANES = 16
NUM_WORKERS = NUM_CORES * NUM_SUBCORES
BPW = BATCH // NUM_WORKERS
IDX_CHUNK = 128
N_CHUNKS = BPW // IDX_CHUNK


def _mf_body(user_hbm, item_hbm, gb_hbm, ub_hbm, ib_hbm, ue_hbm, ie_hbm,
             out_hbm,
             uidx_v, iidx_v, urows_v, irows_v,
             ub_v, ib_v, out_v, gb_v, sem):
    wid = lax.axis_index("s") * NUM_CORES + lax.axis_index("c")
    base = wid * BPW

    pltpu.sync_copy(gb_hbm, gb_v)
    for k in range(N_CHUNKS):
        off = base + k * IDX_CHUNK
        pltpu.sync_copy(user_hbm.at[pl.ds(off, IDX_CHUNK)], uidx_v.at[k])
        pltpu.sync_copy(item_hbm.at[pl.ds(off, IDX_CHUNK)], iidx_v.at[k])

    for k in range(N_CHUNKS):
        dst = pl.ds(k * IDX_CHUNK, IDX_CHUNK)
        pltpu.async_copy(ub_hbm.at[uidx_v.at[k]], ub_v.at[dst], sem)
        pltpu.async_copy(ib_hbm.at[iidx_v.at[k]], ib_v.at[dst], sem)
        pltpu.async_copy(ue_hbm.at[uidx_v.at[k]], urows_v.at[dst], sem)
        pltpu.async_copy(ie_hbm.at[iidx_v.at[k]], irows_v.at[dst], sem)

    pltpu.make_async_copy(ub_hbm.at[pl.ds(0, BPW)], ub_v, sem).wait()
    pltpu.make_async_copy(ib_hbm.at[pl.ds(0, BPW)], ib_v, sem).wait()
    pltpu.make_async_copy(ue_hbm.at[pl.ds(0, BPW)], urows_v, sem).wait()
    pltpu.make_async_copy(ie_hbm.at[pl.ds(0, BPW)], irows_v, sem).wait()

    g = gb_v[...]
    lanes = lax.iota(jnp.int32, LANES)

    def chunk_body(c, _):
        sl = pl.ds(c * LANES, LANES)
        acc = ub_v[sl] + ib_v[sl] + g
        for r_local in range(LANES):
            r = c * LANES + r_local
            u0 = urows_v[r, pl.ds(0, LANES)]
            u1 = urows_v[r, pl.ds(LANES, LANES)]
            i0 = irows_v[r, pl.ds(0, LANES)]
            i1 = irows_v[r, pl.ds(LANES, LANES)]
            s = jnp.sum(u0 * i0 + u1 * i1)
            acc = jnp.where(lanes == r_local, acc + s, acc)
        out_v[sl] = acc
        return _

    lax.fori_loop(0, BPW // LANES, chunk_body, None)

    pltpu.sync_copy(out_v, out_hbm.at[pl.ds(base, BPW)])


_mf_kernel = pl.kernel(
    _mf_body,
    out_type=jax.ShapeDtypeStruct((BATCH,), jnp.float32),
    mesh=plsc.VectorSubcoreMesh(core_axis_name="c", subcore_axis_name="s",
                                num_cores=NUM_CORES,
                                num_subcores=NUM_SUBCORES),
    scratch_types=[
        pltpu.VMEM((N_CHUNKS, IDX_CHUNK), jnp.int32),
        pltpu.VMEM((N_CHUNKS, IDX_CHUNK), jnp.int32),
        pltpu.VMEM((BPW, N_FEATS), jnp.float32),
        pltpu.VMEM((BPW, N_FEATS), jnp.float32),
        pltpu.VMEM((BPW,), jnp.float32),
        pltpu.VMEM((BPW,), jnp.float32),
        pltpu.VMEM((BPW,), jnp.float32),
        pltpu.VMEM((LANES,), jnp.float32),
        pltpu.SemaphoreType.DMA,
    ],
    compiler_params=pltpu.CompilerParams(needs_layout_passes=False,
                                         use_tc_tiling_on_sc=False),
)


@jax.jit
def kernel(user, item, g_bias, u_bias_w, i_bias_w, u_embed_w, i_embed_w):
    gb = jnp.full((LANES,), g_bias, jnp.float32)
    ub = jnp.reshape(u_bias_w, (-1,))
    ib = jnp.reshape(i_bias_w, (-1,))
    return _mf_kernel(user, item, gb, ub, ib, u_embed_w, i_embed_w)
